# Initial kernel scaffold; baseline (speedup 1.0000x reference)
import jax
import jax.numpy as jnp
from jax.experimental import pallas as pl


def kernel(x, edge_index, edge_weight, embed_weight):
    raise NotImplementedError("write your pallas kernel here")


# speedup vs baseline: 3.3250x; 3.3250x over previous
import functools

import jax
import jax.numpy as jnp
from jax import lax
from jax.experimental import pallas as pl
from jax.experimental.pallas import tpu as pltpu
from jax.experimental.pallas import tpu_sc as plsc

N = 50000
D = 64
E = 800000
HALF = N // 2
NS = 16
NC = 2
K = 256
CPT = 196
EPT = CPT * K
EPAD = NS * EPT
DUMMY = HALF
ACCR = 98 * K
ZCH = 98
DCH = 40
DROWS = HALF // DCH


def _layer_body(h, ei, w, zb, out, acc, rowb, colb, wb, msg, idxl, sem):
    c = lax.axis_index("c")
    s = lax.axis_index("s")
    base_row = c * HALF

    for jj in range(7):
        j = s + jj * NS

        @pl.when(j < ZCH)
        def _():
            pltpu.sync_copy(zb, acc.at[pl.ds(j * K, K)])

    plsc.subcore_barrier()

    ebase = s * EPT

    def chunk(k, carry):
        b0 = ebase + k * K
        pltpu.sync_copy(ei.at[0, pl.ds(b0, K)], rowb)
        pltpu.sync_copy(ei.at[1, pl.ds(b0, K)], colb)
        pltpu.sync_copy(w.at[pl.ds(b0, K)], wb)
        pltpu.async_copy(h.at[colb], msg, sem).wait()

        for g in range(K // 16):
            rv = rowb[pl.ds(g * 16, 16)]
            loc = rv - base_row
            inb = (loc >= 0) & (loc < HALF)
            loc = jnp.where(inb, loc, DUMMY)
            idxl[g // 8, pl.ds((g % 8) * 16, 16)] = loc

        def scale(t, u2):
            for u in range(4):
                e0 = t * 4 + u
                wv = plsc.load_gather(wb, [jnp.full((16,), e0, jnp.int32)])
                for q in range(D // 16):
                    msg[e0, pl.ds(q * 16, 16)] = msg[e0, pl.ds(q * 16, 16)] * wv
            return u2

        lax.fori_loop(0, K // 4, scale, 0)

        for b in range(K // 128):
            pltpu.sync_copy(msg.at[pl.ds(b * 128, 128)], acc.at[idxl.at[b]], add=True)
        return carry

    lax.fori_loop(0, CPT, chunk, 0)

    plsc.subcore_barrier()

    for jj in range(3):
        j = s + jj * NS

        @pl.when(j < DCH)
        def _():
            r0 = j * DROWS
            pltpu.sync_copy(acc.at[pl.ds(r0, DROWS)], out.at[pl.ds(base_row + r0, DROWS)])


def _prep_body(emb, xi, out, xb, rows, sem):
    c = lax.axis_index("c")
    s = lax.axis_index("s")
    wid = s * NC + c
    for jj in range(4):
        j = wid + jj * NS * NC

        @pl.when(j < 125)
        def _():
            r0 = j * 400
            pltpu.sync_copy(xi.at[pl.ds(r0, 400)], xb)
            pltpu.async_copy(emb.at[xb], rows, sem).wait()
            pltpu.sync_copy(rows, out.at[pl.ds(r0, 400)])


def _combine_body(a, b, c, d, o):
    o[...] = (a[...] + b[...] + c[...] + d[...]) * 0.25


def _build():
    mesh = plsc.VectorSubcoreMesh(core_axis_name="c", subcore_axis_name="s")
    sc_params = pltpu.CompilerParams(
        use_tc_tiling_on_sc=False, needs_layout_passes=False
    )

    layer = pl.kernel(
        _layer_body,
        out_type=jax.ShapeDtypeStruct((N, D), jnp.float32),
        mesh=mesh,
        compiler_params=sc_params,
        scratch_types=[
            pltpu.VMEM_SHARED((ACCR, D), jnp.float32),
            pltpu.VMEM((K,), jnp.int32),
            pltpu.VMEM((K,), jnp.int32),
            pltpu.VMEM((K,), jnp.float32),
            pltpu.VMEM((K, D), jnp.float32),
            pltpu.VMEM((K // 128, 128), jnp.int32),
            pltpu.SemaphoreType.DMA,
        ],
    )

    prep = pl.kernel(
        _prep_body,
        out_type=jax.ShapeDtypeStruct((N, D), jnp.float32),
        mesh=mesh,
        compiler_params=sc_params,
        scratch_types=[
            pltpu.VMEM((400,), jnp.int32),
            pltpu.VMEM((400, D), jnp.float32),
            pltpu.SemaphoreType.DMA,
        ],
    )

    combine = pl.pallas_call(
        _combine_body,
        out_shape=jax.ShapeDtypeStruct((N, D), jnp.float32),
        grid=(125,),
        in_specs=[pl.BlockSpec((400, D), lambda i: (i, 0))] * 4,
        out_specs=pl.BlockSpec((400, D), lambda i: (i, 0)),
    )
    return prep, layer, combine


@jax.jit
def kernel(x, edge_index, edge_weight, embed_weight):
    prep, layer, combine = _build()
    pad = EPAD - E
    ei = jnp.pad(edge_index.astype(jnp.int32), ((0, 0), (0, pad)))
    ww = jnp.pad(edge_weight, (0, pad))
    zb = jnp.zeros((K, D), jnp.float32)
    h0 = prep(embed_weight, x.astype(jnp.int32))
    h1 = layer(h0, ei, ww, zb)
    h2 = layer(h1, ei, ww, zb)
    h3 = layer(h2, ei, ww, zb)
    return combine(h0, h1, h2, h3)


# speedup vs baseline: 3.9864x; 1.1989x over previous
import functools

import jax
import jax.numpy as jnp
from jax import lax
from jax.experimental import pallas as pl
from jax.experimental.pallas import tpu as pltpu
from jax.experimental.pallas import tpu_sc as plsc

N = 50000
D = 64
E = 800000
HALF = N // 2
NS = 16
NC = 2
K = 224
CPT = 224
EPT = CPT * K
EPAD = NS * EPT
DUMMY = HALF
ACCR = 25008
ZCH = 97
DCH = 40
DROWS = HALF // DCH


def _layer_body(h, ei, w, zb, out, acc, rowb, colb, wb, msg, idxa, idxb, gsem):
    c = lax.axis_index("c")
    s = lax.axis_index("s")
    base_row = c * HALF

    for jj in range(7):
        j = s + jj * NS

        @pl.when(j < ZCH)
        def _():
            pltpu.sync_copy(zb, acc.at[pl.ds(j * 256, 256)])

    @pl.when(s == 0)
    def _():
        pltpu.sync_copy(zb.at[pl.ds(0, 176)], acc.at[pl.ds(ZCH * 256, 176)])

    plsc.subcore_barrier()

    ebase = s * EPT

    def load_edges(k, p):
        b0 = ebase + k * K
        pltpu.sync_copy(ei.at[0, pl.ds(b0, K)], rowb[p])
        pltpu.sync_copy(ei.at[1, pl.ds(b0, K)], colb[p])
        pltpu.sync_copy(w.at[pl.ds(b0, K)], wb[p])

    load_edges(0, 0)
    load_edges(1, 1)
    pltpu.async_copy(h.at[colb[0]], msg[0], gsem[0])

    def pair(j, carry):
        for p in range(2):
            k = j * 2 + p
            q = 1 - p

            @pl.when(k + 1 < CPT)
            def _():
                pltpu.async_copy(h.at[colb[q]], msg[q], gsem[q])

            pltpu.make_async_copy(h.at[colb[p]], msg[p], gsem[p]).wait()

            for g in range(K // 16):
                rv = rowb[p][pl.ds(g * 16, 16)]
                loc = rv - base_row
                inb = (loc >= 0) & (loc < HALF)
                loc = jnp.where(inb, loc, DUMMY)
                if g < 8:
                    idxa[p][0, pl.ds(g * 16, 16)] = loc
                else:
                    idxb[p][0, pl.ds((g - 8) * 16, 16)] = loc

            def scale(t, u2):
                for u in range(4):
                    e0 = t * 4 + u
                    wv = plsc.load_gather(wb[p], [jnp.full((16,), e0, jnp.int32)])
                    for d in range(D // 16):
                        msg[p][e0, pl.ds(d * 16, 16)] = (
                            msg[p][e0, pl.ds(d * 16, 16)] * wv
                        )
                return u2

            lax.fori_loop(0, K // 4, scale, 0)

            pltpu.sync_copy(msg[p].at[pl.ds(0, 128)], acc.at[idxa[p].at[0]], add=True)
            pltpu.sync_copy(msg[p].at[pl.ds(128, 96)], acc.at[idxb[p].at[0]], add=True)

            @pl.when(k + 2 < CPT)
            def _():
                load_edges(k + 2, p)

        return carry

    lax.fori_loop(0, CPT // 2, pair, 0)

    plsc.subcore_barrier()

    for jj in range(3):
        j = s + jj * NS

        @pl.when(j < DCH)
        def _():
            r0 = j * DROWS
            pltpu.sync_copy(acc.at[pl.ds(r0, DROWS)], out.at[pl.ds(base_row + r0, DROWS)])


def _prep_body(emb, xi, out, xb, rows, sem):
    c = lax.axis_index("c")
    s = lax.axis_index("s")
    wid = s * NC + c
    for jj in range(4):
        j = wid + jj * NS * NC

        @pl.when(j < 125)
        def _():
            r0 = j * 400
            pltpu.sync_copy(xi.at[pl.ds(r0, 400)], xb)
            pltpu.async_copy(emb.at[xb], rows, sem).wait()
            pltpu.sync_copy(rows, out.at[pl.ds(r0, 400)])


def _combine_body(a, b, c, d, o):
    o[...] = (a[...] + b[...] + c[...] + d[...]) * 0.25


def _build():
    mesh = plsc.VectorSubcoreMesh(core_axis_name="c", subcore_axis_name="s")
    sc_params = pltpu.CompilerParams(
        use_tc_tiling_on_sc=False, needs_layout_passes=False
    )

    layer = pl.kernel(
        _layer_body,
        out_type=jax.ShapeDtypeStruct((N, D), jnp.float32),
        mesh=mesh,
        compiler_params=sc_params,
        scratch_types=[
            pltpu.VMEM_SHARED((ACCR, D), jnp.float32),
            [pltpu.VMEM((K,), jnp.int32)] * 2,
            [pltpu.VMEM((K,), jnp.int32)] * 2,
            [pltpu.VMEM((K,), jnp.float32)] * 2,
            [pltpu.VMEM((K, D), jnp.float32)] * 2,
            [pltpu.VMEM((1, 128), jnp.int32)] * 2,
            [pltpu.VMEM((1, 96), jnp.int32)] * 2,
            [pltpu.SemaphoreType.DMA] * 2,
        ],
    )

    prep = pl.kernel(
        _prep_body,
        out_type=jax.ShapeDtypeStruct((N, D), jnp.float32),
        mesh=mesh,
        compiler_params=sc_params,
        scratch_types=[
            pltpu.VMEM((400,), jnp.int32),
            pltpu.VMEM((400, D), jnp.float32),
            pltpu.SemaphoreType.DMA,
        ],
    )

    combine = pl.pallas_call(
        _combine_body,
        out_shape=jax.ShapeDtypeStruct((N, D), jnp.float32),
        grid=(125,),
        in_specs=[pl.BlockSpec((400, D), lambda i: (i, 0))] * 4,
        out_specs=pl.BlockSpec((400, D), lambda i: (i, 0)),
    )
    return prep, layer, combine


@jax.jit
def kernel(x, edge_index, edge_weight, embed_weight):
    prep, layer, combine = _build()
    pad = EPAD - E
    ei = jnp.pad(edge_index.astype(jnp.int32), ((0, 0), (0, pad)))
    ww = jnp.pad(edge_weight, (0, pad))
    zb = jnp.zeros((256, D), jnp.float32)
    h0 = prep(embed_weight, x.astype(jnp.int32))
    h1 = layer(h0, ei, ww, zb)
    h2 = layer(h1, ei, ww, zb)
    h3 = layer(h2, ei, ww, zb)
    return combine(h0, h1, h2, h3)


# speedup vs baseline: 5.1726x; 1.2975x over previous
import functools

import jax
import jax.numpy as jnp
from jax import lax
from jax.experimental import pallas as pl
from jax.experimental.pallas import tpu as pltpu
from jax.experimental.pallas import tpu_sc as plsc

N = 50000
D = 64
E = 800000
HALF = N // 2
NS = 16
NC = 2
K = 224
CPT = 224
EPT = CPT * K
EPAD = NS * EPT
DUMMY = HALF
ACCR = 25008
ZCH = 97
DCH = 40
DROWS = HALF // DCH


MSGB = K * D * 4
EDGB = 3 * K * 4


def _layer_body(h, ei, w, zb, out, acc, rowb, colb, wb, msg, idxa, idxb, gsem, ssem, esem):
    c = lax.axis_index("c")
    s = lax.axis_index("s")
    base_row = c * HALF

    for jj in range(7):
        j = s + jj * NS

        @pl.when(j < ZCH)
        def _():
            pltpu.sync_copy(zb, acc.at[pl.ds(j * 256, 256)])

    @pl.when(s == 0)
    def _():
        pltpu.sync_copy(zb.at[pl.ds(0, 176)], acc.at[pl.ds(ZCH * 256, 176)])

    plsc.subcore_barrier()

    ebase = s * EPT

    def wait_msg(sem, buf):
        pltpu.make_async_copy(h.at[pl.ds(0, K)], buf, sem).wait()

    def wait_edges(sem, p):
        pltpu.make_async_copy(ei.at[0, pl.ds(0, K)], rowb[p], sem).wait()
        pltpu.make_async_copy(ei.at[1, pl.ds(0, K)], colb[p], sem).wait()
        pltpu.make_async_copy(w.at[pl.ds(0, K)], wb[p], sem).wait()

    for p0 in range(2):
        b0 = ebase + p0 * K
        pltpu.sync_copy(ei.at[0, pl.ds(b0, K)], rowb[p0])
        pltpu.sync_copy(ei.at[1, pl.ds(b0, K)], colb[p0])
        pltpu.sync_copy(w.at[pl.ds(b0, K)], wb[p0])
    pltpu.async_copy(h.at[colb[0]], msg[0], gsem[0])

    def pair(j, carry):
        for p in range(2):
            k = j * 2 + p
            q = 1 - p

            @pl.when(k + 1 < CPT)
            def _():
                @pl.when(k >= 1)
                def _():
                    wait_msg(ssem[q], msg[q])
                    wait_edges(esem[q], q)

                pltpu.async_copy(h.at[colb[q]], msg[q], gsem[q])

            wait_msg(gsem[p], msg[p])

            for g in range(K // 16):
                rv = rowb[p][pl.ds(g * 16, 16)]
                loc = rv - base_row
                inb = (loc >= 0) & (loc < HALF)
                loc = jnp.where(inb, loc, DUMMY)
                if g < 8:
                    idxa[p][0, pl.ds(g * 16, 16)] = loc
                else:
                    idxb[p][0, pl.ds((g - 8) * 16, 16)] = loc

            @plsc.parallel_loop(0, K, 1, unroll=8)
            def _(e0):
                wv = plsc.load_gather(wb[p], [jnp.full((16,), e0, jnp.int32)])
                for d in range(D // 16):
                    msg[p][e0, pl.ds(d * 16, 16)] = msg[p][e0, pl.ds(d * 16, 16)] * wv

            pltpu.async_copy(msg[p].at[pl.ds(0, 128)], acc.at[idxa[p].at[0]], ssem[p], add=True)
            pltpu.async_copy(msg[p].at[pl.ds(128, 96)], acc.at[idxb[p].at[0]], ssem[p], add=True)

            @pl.when(k + 2 < CPT)
            def _():
                b2 = ebase + (k + 2) * K
                pltpu.async_copy(ei.at[0, pl.ds(b2, K)], rowb[p], esem[p])
                pltpu.async_copy(ei.at[1, pl.ds(b2, K)], colb[p], esem[p])
                pltpu.async_copy(w.at[pl.ds(b2, K)], wb[p], esem[p])

        return carry

    lax.fori_loop(0, CPT // 2, pair, 0)

    wait_msg(ssem[0], msg[0])
    wait_msg(ssem[1], msg[1])

    plsc.subcore_barrier()

    for jj in range(3):
        j = s + jj * NS

        @pl.when(j < DCH)
        def _():
            r0 = j * DROWS
            pltpu.sync_copy(acc.at[pl.ds(r0, DROWS)], out.at[pl.ds(base_row + r0, DROWS)])


def _prep_body(emb, xi, out, xb, rows, sem):
    c = lax.axis_index("c")
    s = lax.axis_index("s")
    wid = s * NC + c
    for jj in range(4):
        j = wid + jj * NS * NC

        @pl.when(j < 125)
        def _():
            r0 = j * 400
            pltpu.sync_copy(xi.at[pl.ds(r0, 400)], xb)
            pltpu.async_copy(emb.at[xb], rows, sem).wait()
            pltpu.sync_copy(rows, out.at[pl.ds(r0, 400)])


def _combine_body(a, b, c, d, o):
    o[...] = (a[...] + b[...] + c[...] + d[...]) * 0.25


def _build():
    mesh = plsc.VectorSubcoreMesh(core_axis_name="c", subcore_axis_name="s")
    sc_params = pltpu.CompilerParams(
        use_tc_tiling_on_sc=False, needs_layout_passes=False
    )

    layer = pl.kernel(
        _layer_body,
        out_type=jax.ShapeDtypeStruct((N, D), jnp.float32),
        mesh=mesh,
        compiler_params=sc_params,
        scratch_types=[
            pltpu.VMEM_SHARED((ACCR, D), jnp.float32),
            [pltpu.VMEM((K,), jnp.int32)] * 2,
            [pltpu.VMEM((K,), jnp.int32)] * 2,
            [pltpu.VMEM((K,), jnp.float32)] * 2,
            [pltpu.VMEM((K, D), jnp.float32)] * 2,
            [pltpu.VMEM((1, 128), jnp.int32)] * 2,
            [pltpu.VMEM((1, 96), jnp.int32)] * 2,
            [pltpu.SemaphoreType.DMA] * 2,
            [pltpu.SemaphoreType.DMA] * 2,
            [pltpu.SemaphoreType.DMA] * 2,
        ],
    )

    prep = pl.kernel(
        _prep_body,
        out_type=jax.ShapeDtypeStruct((N, D), jnp.float32),
        mesh=mesh,
        compiler_params=sc_params,
        scratch_types=[
            pltpu.VMEM((400,), jnp.int32),
            pltpu.VMEM((400, D), jnp.float32),
            pltpu.SemaphoreType.DMA,
        ],
    )

    combine = pl.pallas_call(
        _combine_body,
        out_shape=jax.ShapeDtypeStruct((N, D), jnp.float32),
        grid=(125,),
        in_specs=[pl.BlockSpec((400, D), lambda i: (i, 0))] * 4,
        out_specs=pl.BlockSpec((400, D), lambda i: (i, 0)),
    )
    return prep, layer, combine


@jax.jit
def kernel(x, edge_index, edge_weight, embed_weight):
    prep, layer, combine = _build()
    pad = EPAD - E
    ei = jnp.pad(edge_index.astype(jnp.int32), ((0, 0), (0, pad)))
    ww = jnp.pad(edge_weight, (0, pad))
    zb = jnp.zeros((256, D), jnp.float32)
    h0 = prep(embed_weight, x.astype(jnp.int32))
    h1 = layer(h0, ei, ww, zb)
    h2 = layer(h1, ei, ww, zb)
    h3 = layer(h2, ei, ww, zb)
    return combine(h0, h1, h2, h3)


# speedup vs baseline: 7.9785x; 1.5425x over previous
import jax
import jax.numpy as jnp
from jax import lax
from jax.experimental import pallas as pl
from jax.experimental.pallas import tpu as pltpu
from jax.experimental.pallas import tpu_sc as plsc

N = 50000
D = 64
E = 800000
HALF = N // 2
NS = 16
NC = 2
K = 224
CPT = 224
EPT = CPT * K
EPAD = NS * EPT
DUMMY = HALF
ACCR = 25008
ZCH = 97
DCH = 40
DROWS = HALF // DCH
SEG = EPT
NSEG = NC * NS
STG = 912
FL = 2 * K


def _part_body(ei, w, colp, wp, locp, counts, rowb, colb, wb, stc, stw, stl, cntb, esem):
    c = lax.axis_index("c")
    s = lax.axis_index("s")
    base_row = c * HALF
    wid = c * NS + s
    seg = wid * SEG
    ebase = s * EPT

    def wait_edges(p):
        pltpu.make_async_copy(ei.at[0, pl.ds(0, K)], rowb[p], esem[p]).wait()
        pltpu.make_async_copy(ei.at[1, pl.ds(0, K)], colb[p], esem[p]).wait()
        pltpu.make_async_copy(w.at[pl.ds(0, K)], wb[p], esem[p]).wait()

    def issue_edges(k, p):
        b0 = ebase + k * K
        pltpu.async_copy(ei.at[0, pl.ds(b0, K)], rowb[p], esem[p])
        pltpu.async_copy(ei.at[1, pl.ds(b0, K)], colb[p], esem[p])
        pltpu.async_copy(w.at[pl.ds(b0, K)], wb[p], esem[p])

    pltpu.sync_copy(ei.at[0, pl.ds(ebase, K)], rowb[0])
    pltpu.sync_copy(ei.at[1, pl.ds(ebase, K)], colb[0])
    pltpu.sync_copy(w.at[pl.ds(ebase, K)], wb[0])
    issue_edges(1, 1)

    def pair(j, carry):
        sp, wof = carry
        for p in range(2):
            k = j * 2 + p

            @pl.when(k >= 1)
            def _():
                wait_edges(p)

            for g in range(K // 16):
                rv = rowb[p][pl.ds(g * 16, 16)]
                cv = colb[p][pl.ds(g * 16, 16)]
                wv = wb[p][pl.ds(g * 16, 16)]
                loc = rv - base_row
                inb = (loc >= 0) & (loc < HALF)
                plsc.store_compressed(stc.at[pl.ds(sp, 16)], cv, mask=inb)
                plsc.store_compressed(stw.at[pl.ds(sp, 16)], wv, mask=inb)
                plsc.store_compressed(stl.at[pl.ds(sp, 16)], loc, mask=inb)
                sp = sp + jnp.sum(inb.astype(jnp.int32))

            @pl.when(k + 2 < CPT)
            def _():
                issue_edges(k + 2, p)

            full = sp >= FL

            @pl.when(full)
            def _():
                fo = pl.multiple_of(seg + wof, 8)
                pltpu.sync_copy(stc.at[pl.ds(0, FL)], colp.at[pl.ds(fo, FL)])
                pltpu.sync_copy(stw.at[pl.ds(0, FL)], wp.at[pl.ds(fo, FL)])
                pltpu.sync_copy(stl.at[pl.ds(0, FL)], locp.at[pl.ds(fo, FL)])
                nmv = (sp - FL + 15) // 16

                def mv(i, u):
                    stc[pl.ds(i * 16, 16)] = stc[pl.ds(FL + i * 16, 16)]
                    stw[pl.ds(i * 16, 16)] = stw[pl.ds(FL + i * 16, 16)]
                    stl[pl.ds(i * 16, 16)] = stl[pl.ds(FL + i * 16, 16)]
                    return u

                lax.fori_loop(0, nmv, mv, 0)

            sp = jnp.where(full, sp - FL, sp)
            wof = jnp.where(full, wof + FL, wof)
        return sp, wof

    sp, wof = lax.fori_loop(0, CPT // 2, pair, (jnp.int32(0), jnp.int32(0)))

    cnt = wof + sp
    target = jnp.maximum(FL, ((cnt + FL - 1) // FL) * FL)
    npad16 = (target - cnt + 15) // 16
    zc = jnp.zeros((16,), jnp.int32)
    zw = jnp.zeros((16,), jnp.float32)
    dl = jnp.full((16,), DUMMY, jnp.int32)

    def padb(i, u):
        stc[pl.ds(sp + i * 16, 16)] = zc
        stw[pl.ds(sp + i * 16, 16)] = zw
        stl[pl.ds(sp + i * 16, 16)] = dl
        return u

    lax.fori_loop(0, npad16, padb, 0)

    @pl.when(target - wof > 0)
    def _():
        fo = pl.multiple_of(seg + wof, 8)
        pltpu.sync_copy(stc.at[pl.ds(0, FL)], colp.at[pl.ds(fo, FL)])
        pltpu.sync_copy(stw.at[pl.ds(0, FL)], wp.at[pl.ds(fo, FL)])
        pltpu.sync_copy(stl.at[pl.ds(0, FL)], locp.at[pl.ds(fo, FL)])

    cntb[pl.ds(0, 16)] = jnp.zeros((16,), jnp.int32) + target
    pltpu.sync_copy(cntb, counts.at[pl.ds(wid * 16, 16)])


def _layer_body(h, colp, wp, locp, counts, zb, out, acc, colb, wb, msg, idxa, idxb, cntv, gsem, ssem, esem):
    c = lax.axis_index("c")
    s = lax.axis_index("s")
    base_row = c * HALF
    wid = c * NS + s
    seg = wid * SEG

    for jj in range(7):
        j = s + jj * NS

        @pl.when(j < ZCH)
        def _():
            pltpu.sync_copy(zb, acc.at[pl.ds(j * 256, 256)])

    @pl.when(s == 0)
    def _():
        pltpu.sync_copy(zb.at[pl.ds(0, 176)], acc.at[pl.ds(ZCH * 256, 176)])

    plsc.subcore_barrier()

    pltpu.sync_copy(counts.at[pl.ds(wid * 16, 16)], cntv)
    cnt = jnp.max(cntv[pl.ds(0, 16)])
    nb = cnt // K

    def wait_msg(sem, buf):
        pltpu.make_async_copy(h.at[pl.ds(0, K)], buf, sem).wait()

    def wait_edges(p):
        pltpu.make_async_copy(colp.at[pl.ds(0, K)], colb[p], esem[p]).wait()
        pltpu.make_async_copy(wp.at[pl.ds(0, K)], wb[p], esem[p]).wait()
        pltpu.make_async_copy(locp.at[pl.ds(0, 128)], idxa[p].at[0], esem[p]).wait()
        pltpu.make_async_copy(locp.at[pl.ds(0, 96)], idxb[p].at[0], esem[p]).wait()

    def load_edges_sync(k, p):
        koff = seg + k * K
        pltpu.sync_copy(colp.at[pl.ds(koff, K)], colb[p])
        pltpu.sync_copy(wp.at[pl.ds(koff, K)], wb[p])
        pltpu.sync_copy(locp.at[pl.ds(koff, 128)], idxa[p].at[0])
        pltpu.sync_copy(locp.at[pl.ds(koff + 128, 96)], idxb[p].at[0])

    def issue_edges(k, p):
        koff = seg + k * K
        pltpu.async_copy(colp.at[pl.ds(koff, K)], colb[p], esem[p])
        pltpu.async_copy(wp.at[pl.ds(koff, K)], wb[p], esem[p])
        pltpu.async_copy(locp.at[pl.ds(koff, 128)], idxa[p].at[0], esem[p])
        pltpu.async_copy(locp.at[pl.ds(koff + 128, 96)], idxb[p].at[0], esem[p])

    load_edges_sync(0, 0)
    load_edges_sync(1, 1)
    pltpu.async_copy(h.at[colb[0]], msg[0], gsem[0])

    def pair(j, carry):
        for p in range(2):
            k = j * 2 + p
            q = 1 - p

            @pl.when(k + 1 < nb)
            def _():
                @pl.when(k >= 1)
                def _():
                    wait_msg(ssem[q], msg[q])
                    wait_edges(q)

                pltpu.async_copy(h.at[colb[q]], msg[q], gsem[q])

            wait_msg(gsem[p], msg[p])

            @plsc.parallel_loop(0, K, 1, unroll=8)
            def _(e0):
                wv = plsc.load_gather(wb[p], [jnp.full((16,), e0, jnp.int32)])
                for d in range(D // 16):
                    msg[p][e0, pl.ds(d * 16, 16)] = msg[p][e0, pl.ds(d * 16, 16)] * wv

            pltpu.async_copy(msg[p].at[pl.ds(0, 128)], acc.at[idxa[p].at[0]], ssem[p], add=True)
            pltpu.async_copy(msg[p].at[pl.ds(128, 96)], acc.at[idxb[p].at[0]], ssem[p], add=True)

            @pl.when(k + 2 < nb)
            def _():
                issue_edges(k + 2, p)

        return carry

    lax.fori_loop(0, nb // 2, pair, 0)

    wait_msg(ssem[0], msg[0])
    wait_msg(ssem[1], msg[1])

    plsc.subcore_barrier()

    for jj in range(3):
        j = s + jj * NS

        @pl.when(j < DCH)
        def _():
            r0 = j * DROWS
            pltpu.sync_copy(acc.at[pl.ds(r0, DROWS)], out.at[pl.ds(base_row + r0, DROWS)])


def _prep_body(emb, xi, out, xb, rows, sem):
    c = lax.axis_index("c")
    s = lax.axis_index("s")
    wid = s * NC + c
    for jj in range(4):
        j = wid + jj * NS * NC

        @pl.when(j < 125)
        def _():
            r0 = j * 400
            pltpu.sync_copy(xi.at[pl.ds(r0, 400)], xb)
            pltpu.async_copy(emb.at[xb], rows, sem).wait()
            pltpu.sync_copy(rows, out.at[pl.ds(r0, 400)])


def _combine_body(a, b, c, d, o):
    o[...] = (a[...] + b[...] + c[...] + d[...]) * 0.25


def _build():
    mesh = plsc.VectorSubcoreMesh(core_axis_name="c", subcore_axis_name="s")
    sc_params = pltpu.CompilerParams(
        use_tc_tiling_on_sc=False, needs_layout_passes=False
    )

    part = pl.kernel(
        _part_body,
        out_type=(
            jax.ShapeDtypeStruct((NSEG * SEG,), jnp.int32),
            jax.ShapeDtypeStruct((NSEG * SEG,), jnp.float32),
            jax.ShapeDtypeStruct((NSEG * SEG,), jnp.int32),
            jax.ShapeDtypeStruct((NSEG * 16,), jnp.int32),
        ),
        mesh=mesh,
        compiler_params=sc_params,
        scratch_types=[
            [pltpu.VMEM((K,), jnp.int32)] * 2,
            [pltpu.VMEM((K,), jnp.int32)] * 2,
            [pltpu.VMEM((K,), jnp.float32)] * 2,
            pltpu.VMEM((STG,), jnp.int32),
            pltpu.VMEM((STG,), jnp.float32),
            pltpu.VMEM((STG,), jnp.int32),
            pltpu.VMEM((16,), jnp.int32),
            [pltpu.SemaphoreType.DMA] * 2,
        ],
    )

    layer = pl.kernel(
        _layer_body,
        out_type=jax.ShapeDtypeStruct((N, D), jnp.float32),
        mesh=mesh,
        compiler_params=sc_params,
        scratch_types=[
            pltpu.VMEM_SHARED((ACCR, D), jnp.float32),
            [pltpu.VMEM((K,), jnp.int32)] * 2,
            [pltpu.VMEM((K,), jnp.float32)] * 2,
            [pltpu.VMEM((K, D), jnp.float32)] * 2,
            [pltpu.VMEM((1, 128), jnp.int32)] * 2,
            [pltpu.VMEM((1, 96), jnp.int32)] * 2,
            pltpu.VMEM((16,), jnp.int32),
            [pltpu.SemaphoreType.DMA] * 2,
            [pltpu.SemaphoreType.DMA] * 2,
            [pltpu.SemaphoreType.DMA] * 2,
        ],
    )

    prep = pl.kernel(
        _prep_body,
        out_type=jax.ShapeDtypeStruct((N, D), jnp.float32),
        mesh=mesh,
        compiler_params=sc_params,
        scratch_types=[
            pltpu.VMEM((400,), jnp.int32),
            pltpu.VMEM((400, D), jnp.float32),
            pltpu.SemaphoreType.DMA,
        ],
    )

    combine = pl.pallas_call(
        _combine_body,
        out_shape=jax.ShapeDtypeStruct((N, D), jnp.float32),
        grid=(125,),
        in_specs=[pl.BlockSpec((400, D), lambda i: (i, 0))] * 4,
        out_specs=pl.BlockSpec((400, D), lambda i: (i, 0)),
    )
    return prep, part, layer, combine


@jax.jit
def kernel(x, edge_index, edge_weight, embed_weight):
    prep, part, layer, combine = _build()
    pad = EPAD - E
    ei = jnp.pad(edge_index.astype(jnp.int32), ((0, 0), (0, pad)))
    ww = jnp.pad(edge_weight, (0, pad))
    zb = jnp.zeros((256, D), jnp.float32)
    h0 = prep(embed_weight, x.astype(jnp.int32))
    colp, wp, locp, counts = part(ei, ww)
    h1 = layer(h0, colp, wp, locp, counts, zb)
    h2 = layer(h1, colp, wp, locp, counts, zb)
    h3 = layer(h2, colp, wp, locp, counts, zb)
    return combine(h0, h1, h2, h3)
